# baseline (speedup 1.0000x reference)
import functools

import jax
import jax.numpy as jnp
from jax import lax
from jax.experimental import pallas as pl
from jax.experimental.pallas import tpu as pltpu
from jax.experimental.pallas import tpu_sc as plsc

N = 10000
D = 128
G = 64
C = 16
E = 320000

NC = 2
NS = 16
NW = NC * NS

CHUNK = 64
EP = 327680
CPW = EP // (NW * CHUNK)
RPT = 624
RPT_LAST = N - (NS - 1) * RPT

BR = 400
NBLK = N // BR


def _sc_mesh():
    return plsc.VectorSubcoreMesh(core_axis_name="c", subcore_axis_name="s")


def _sc_segsum(y, ei, w, zrows, nr):
    NB = 5
    if nr == N:
        ranges = [(si * RPT, RPT) for si in range(NS - 1)] + [((NS - 1) * RPT, RPT_LAST)]
    else:
        assert nr % 8 == 0 and nr // 8 <= NS
        ranges = [(si * 8, 8) if si < nr // 8 else (0, 0) for si in range(NS)]

    @functools.partial(
        pl.kernel,
        out_type=jax.ShapeDtypeStruct((NC, nr, D), jnp.float32),
        mesh=_sc_mesh(),
        scratch_types=(
            [pltpu.VMEM_SHARED((nr, D), jnp.float32)]
            + [pltpu.VMEM((CHUNK,), jnp.int32) for _ in range(NB)]
            + [pltpu.VMEM((CHUNK,), jnp.int32) for _ in range(NB)]
            + [pltpu.VMEM((CHUNK,), jnp.float32) for _ in range(NB)]
            + [pltpu.VMEM((CHUNK, D), jnp.float32) for _ in range(NB)]
            + [pltpu.SemaphoreType.DMA for _ in range(3 * NB)]
        ),
    )
    def run(y_hbm, src_hbm, dst_hbm, w_hbm, zr_hbm, z_hbm, acc_sh, *scr):
        sidxs = scr[0:NB]
        didxs = scr[NB:2 * NB]
        ws = scr[2 * NB:3 * NB]
        rows = scr[3 * NB:4 * NB]
        isem = scr[4 * NB:5 * NB]
        gsem = scr[5 * NB:6 * NB]
        ssem = scr[6 * NB:7 * NB]
        c = lax.axis_index("c")
        s = lax.axis_index("s")
        for si, (rb, rs) in enumerate(ranges):
            if rs:
                @pl.when(s == si)
                def _(rb=rb, rs=rs):
                    pltpu.sync_copy(zr_hbm.at[pl.ds(0, rs)],
                                    acc_sh.at[pl.ds(rb, rs)])

        plsc.subcore_barrier()
        wid = c * NS + s
        cbase = wid * CPW

        def start_idx(ci, b):
            base = (cbase + ci) * CHUNK
            pltpu.async_copy(src_hbm.at[pl.ds(base, CHUNK)], sidxs[b], isem[b])
            pltpu.async_copy(dst_hbm.at[pl.ds(base, CHUNK)], didxs[b], isem[b])
            pltpu.async_copy(w_hbm.at[pl.ds(base, CHUNK)], ws[b], isem[b])

        def wait_idx(b):
            pltpu.make_async_copy(src_hbm.at[pl.ds(0, CHUNK)], sidxs[b], isem[b]).wait()
            pltpu.make_async_copy(dst_hbm.at[pl.ds(0, CHUNK)], didxs[b], isem[b]).wait()
            pltpu.make_async_copy(w_hbm.at[pl.ds(0, CHUNK)], ws[b], isem[b]).wait()

        def start_gather(b):
            pltpu.async_copy(y_hbm.at[sidxs[b]], rows[b], gsem[b])

        def wait_gather(b):
            pltpu.make_async_copy(y_hbm.at[sidxs[b]], rows[b], gsem[b]).wait()

        def start_scatter(b):
            pltpu.async_copy(rows[b], acc_sh.at[didxs[b]], ssem[b], add=True)

        def wait_scatter(b):
            pltpu.make_async_copy(rows[b], acc_sh.at[didxs[b]], ssem[b]).wait()

        def scale(b):
            @pl.loop(0, CHUNK // 16)
            def _grp(k):
                wvec = ws[b][pl.ds(k * 16, 16)]
                for j in range(16):
                    wt = wvec[j]
                    r = k * 16 + j
                    for f in range(D // 16):
                        sl = pl.ds(f * 16, 16)
                        rows[b][r, sl] = rows[b][r, sl] * wt

        for b in range(NB):
            start_idx(b, b)
        for b in range(NB):
            wait_idx(b)
            start_gather(b)

        @pl.loop(0, CPW - NB, step=NB)
        def _super(ci0):
            for b in range(NB):
                wait_gather(b)
                scale(b)
                start_scatter(b)
            for b in range(NB):
                wait_scatter(b)
                start_idx(ci0 + NB + b, b)
            for b in range(NB):
                wait_idx(b)
                start_gather(b)

        for b in range(NB):
            wait_gather(b)
            scale(b)
            start_scatter(b)
        for b in range(NB):
            wait_scatter(b)

        plsc.subcore_barrier()

        for si, (rb, rs) in enumerate(ranges):
            if rs:
                @pl.when(s == si)
                def _(rb=rb, rs=rs):
                    pltpu.sync_copy(acc_sh.at[pl.ds(rb, rs)],
                                    z_hbm.at[c, pl.ds(rb, rs)])

    return run(y, ei[0], ei[1], w, zrows)


def _tc_matmul(x, W):
    def body(x_ref, w_ref, o_ref):
        o_ref[...] = jnp.dot(x_ref[...], w_ref[...],
                             preferred_element_type=jnp.float32)

    return pl.pallas_call(
        body,
        grid=(NBLK,),
        in_specs=[pl.BlockSpec((BR, D), lambda i: (i, 0)),
                  pl.BlockSpec((D, D), lambda i: (0, 0))],
        out_specs=pl.BlockSpec((BR, D), lambda i: (i, 0)),
        out_shape=jax.ShapeDtypeStruct((N, D), jnp.float32),
    )(x, W)


def _tc_layer(z0, z1, xin, b, Wroot, Wnext):
    def body(z0_ref, z1_ref, x_ref, b_ref, wr_ref, wn_ref, h_ref, y_ref):
        acc = (z0_ref[...] + z1_ref[...] + b_ref[...]
               + jnp.dot(x_ref[...], wr_ref[...],
                         preferred_element_type=jnp.float32))
        h = jnp.maximum(acc, 0.0)
        h_ref[...] = h
        y_ref[...] = jnp.dot(h, wn_ref[...], preferred_element_type=jnp.float32)

    return pl.pallas_call(
        body,
        grid=(NBLK,),
        in_specs=[pl.BlockSpec((BR, D), lambda i: (i, 0)),
                  pl.BlockSpec((BR, D), lambda i: (i, 0)),
                  pl.BlockSpec((BR, D), lambda i: (i, 0)),
                  pl.BlockSpec((1, D), lambda i: (0, 0)),
                  pl.BlockSpec((D, D), lambda i: (0, 0)),
                  pl.BlockSpec((D, D), lambda i: (0, 0))],
        out_specs=[pl.BlockSpec((BR, D), lambda i: (i, 0)),
                   pl.BlockSpec((BR, D), lambda i: (i, 0))],
        out_shape=[jax.ShapeDtypeStruct((N, D), jnp.float32),
                   jax.ShapeDtypeStruct((N, D), jnp.float32)],
    )(z0, z1, xin, b, Wroot, Wnext)


def _tc_layer3(z0, z1, h1, b2, W2root, W3rel, W3root, Wlin):
    def body(z0_ref, z1_ref, h1_ref, b_ref, w2r_ref, w3a_ref, w3b_ref,
             wl_ref, y3_ref, r_ref):
        acc = (z0_ref[...] + z1_ref[...] + b_ref[...]
               + jnp.dot(h1_ref[...], w2r_ref[...],
                         preferred_element_type=jnp.float32))
        h2 = jnp.maximum(acc, 0.0)
        y3_ref[...] = jnp.dot(h2, w3a_ref[...], preferred_element_type=jnp.float32)
        wr = jnp.dot(w3b_ref[...], wl_ref[...], preferred_element_type=jnp.float32)
        r_ref[...] = jnp.dot(h2, wr, preferred_element_type=jnp.float32)

    return pl.pallas_call(
        body,
        grid=(NBLK,),
        in_specs=[pl.BlockSpec((BR, D), lambda i: (i, 0)),
                  pl.BlockSpec((BR, D), lambda i: (i, 0)),
                  pl.BlockSpec((BR, D), lambda i: (i, 0)),
                  pl.BlockSpec((1, D), lambda i: (0, 0)),
                  pl.BlockSpec((D, D), lambda i: (0, 0)),
                  pl.BlockSpec((D, D), lambda i: (0, 0)),
                  pl.BlockSpec((D, D), lambda i: (0, 0)),
                  pl.BlockSpec((D, C), lambda i: (0, 0))],
        out_specs=[pl.BlockSpec((BR, D), lambda i: (i, 0)),
                   pl.BlockSpec((BR, C), lambda i: (i, 0))],
        out_shape=[jax.ShapeDtypeStruct((N, D), jnp.float32),
                   jax.ShapeDtypeStruct((N, C), jnp.float32)],
    )(z0, z1, h1, b2, W2root, W3rel, W3root, Wlin)


def _tc_final(z30, z31, batch2, r, b3, Wlin, blin):
    def body(z30_ref, z31_ref, b_ref, r_ref, b3_ref, wl_ref, bl_ref, o_ref,
             acc128_ref, acc_ref, cnt_ref):
        i = pl.program_id(0)

        @pl.when(i == 0)
        def _():
            acc128_ref[...] = jnp.zeros_like(acc128_ref)
            acc_ref[...] = jnp.zeros_like(acc_ref)
            cnt_ref[...] = jnp.zeros_like(cnt_ref)

        bv = b_ref[...]
        onehot = (bv == lax.broadcasted_iota(jnp.int32, (BR, G), 1)
                  ).astype(jnp.float32)
        acc128_ref[...] += lax.dot_general(
            onehot, z30_ref[...] + z31_ref[...], (((0,), (0,)), ((), ())),
            preferred_element_type=jnp.float32)
        acc_ref[...] += lax.dot_general(
            onehot, r_ref[...], (((0,), (0,)), ((), ())),
            preferred_element_type=jnp.float32)
        cnt_ref[...] += lax.dot_general(
            onehot, jnp.ones((BR, 1), jnp.float32), (((0,), (0,)), ((), ())),
            preferred_element_type=jnp.float32)

        @pl.when(i == NBLK - 1)
        def _():
            cnt = cnt_ref[...]
            b3w = jnp.dot(b3_ref[...], wl_ref[...],
                          preferred_element_type=jnp.float32)
            edge_term = jnp.dot(acc128_ref[...], wl_ref[...],
                                preferred_element_type=jnp.float32)
            tot = edge_term + acc_ref[...] + cnt * b3w
            logits = tot / jnp.maximum(cnt, 1.0) + bl_ref[...]
            m = jnp.max(logits, axis=-1, keepdims=True)
            ex = jnp.exp(logits - m)
            o_ref[...] = ex / jnp.sum(ex, axis=-1, keepdims=True)

    return pl.pallas_call(
        body,
        grid=(NBLK,),
        in_specs=[pl.BlockSpec((BR, D), lambda i: (i, 0)),
                  pl.BlockSpec((BR, D), lambda i: (i, 0)),
                  pl.BlockSpec((BR, 1), lambda i: (i, 0)),
                  pl.BlockSpec((BR, C), lambda i: (i, 0)),
                  pl.BlockSpec((1, D), lambda i: (0, 0)),
                  pl.BlockSpec((D, C), lambda i: (0, 0)),
                  pl.BlockSpec((1, C), lambda i: (0, 0))],
        out_specs=pl.BlockSpec((G, C), lambda i: (0, 0)),
        out_shape=jax.ShapeDtypeStruct((G, C), jnp.float32),
        scratch_shapes=[pltpu.VMEM((G, D), jnp.float32),
                        pltpu.VMEM((G, C), jnp.float32),
                        pltpu.VMEM((G, 1), jnp.float32)],
    )(z30, z31, batch2, r, b3, Wlin, blin)


def kernel(x, edge_attr, edge_index, batch,
           W1_rel, b1_rel, W1_root,
           W2_rel, b2_rel, W2_root,
           W3_rel, b3_rel, W3_root,
           W_lin, b_lin):
    pad = EP - E
    pad_idx = jnp.arange(pad, dtype=jnp.int32) % N
    src = jnp.concatenate([edge_index[0], pad_idx])
    dst = jnp.concatenate([edge_index[1], pad_idx])
    ei = jnp.stack([src, dst])
    w = jnp.concatenate([edge_attr, jnp.zeros((pad,), jnp.float32)])
    zrows = jnp.zeros((RPT_LAST, D), jnp.float32)
    batch2 = batch.reshape(N, 1)

    y1 = _tc_matmul(x, W1_rel)
    z1 = _sc_segsum(y1, ei, w, zrows, N)
    h1, y2 = _tc_layer(z1[0], z1[1], x, b1_rel.reshape(1, D), W1_root, W2_rel)
    z2 = _sc_segsum(y2, ei, w, zrows, N)
    y3, r = _tc_layer3(z2[0], z2[1], h1, b2_rel.reshape(1, D), W2_root,
                       W3_rel, W3_root, W_lin)
    z3 = _sc_segsum(y3, ei, w, zrows, N)
    return _tc_final(z3[0], z3[1], batch2, r, b3_rel.reshape(1, D), W_lin,
                     b_lin.reshape(1, C))


# speedup vs baseline: 1.0236x; 1.0236x over previous
import functools

import jax
import jax.numpy as jnp
from jax import lax
from jax.experimental import pallas as pl
from jax.experimental.pallas import tpu as pltpu
from jax.experimental.pallas import tpu_sc as plsc

N = 10000
D = 128
G = 64
C = 16
E = 320000

NC = 2
NS = 16
NW = NC * NS

CHUNK = 64
EP = 327680
CPW = EP // (NW * CHUNK)
RPT = 624
RPT_LAST = N - (NS - 1) * RPT

BR = 400
NBLK = N // BR


def _sc_mesh():
    return plsc.VectorSubcoreMesh(core_axis_name="c", subcore_axis_name="s")


def _sc_segsum(y, ei, w, zrows, nr):
    NB = 4
    if nr == N:
        ranges = [(si * RPT, RPT) for si in range(NS - 1)] + [((NS - 1) * RPT, RPT_LAST)]
    else:
        assert nr % 8 == 0 and nr // 8 <= NS
        ranges = [(si * 8, 8) if si < nr // 8 else (0, 0) for si in range(NS)]

    @functools.partial(
        pl.kernel,
        out_type=jax.ShapeDtypeStruct((NC, nr, D), jnp.float32),
        mesh=_sc_mesh(),
        scratch_types=(
            [pltpu.VMEM_SHARED((nr, D), jnp.float32)]
            + [pltpu.VMEM((CHUNK,), jnp.int32) for _ in range(NB)]
            + [pltpu.VMEM((CHUNK,), jnp.int32) for _ in range(NB)]
            + [pltpu.VMEM((CHUNK,), jnp.float32) for _ in range(NB)]
            + [pltpu.VMEM((CHUNK, D), jnp.float32) for _ in range(NB)]
            + [pltpu.SemaphoreType.DMA for _ in range(4 * NB)]
        ),
    )
    def run(y_hbm, src_hbm, dst_hbm, w_hbm, zr_hbm, z_hbm, acc_sh, *scr):
        sidxs = scr[0:NB]
        didxs = scr[NB:2 * NB]
        ws = scr[2 * NB:3 * NB]
        rows = scr[3 * NB:4 * NB]
        swsem = scr[4 * NB:5 * NB]
        dsem = scr[5 * NB:6 * NB]
        gsem = scr[6 * NB:7 * NB]
        ssem = scr[7 * NB:8 * NB]
        c = lax.axis_index("c")
        s = lax.axis_index("s")
        for si, (rb, rs) in enumerate(ranges):
            if rs:
                @pl.when(s == si)
                def _(rb=rb, rs=rs):
                    pltpu.sync_copy(zr_hbm.at[pl.ds(0, rs)],
                                    acc_sh.at[pl.ds(rb, rs)])

        plsc.subcore_barrier()
        wid = c * NS + s
        cbase = wid * CPW

        def start_sw(ci, b):
            base = (cbase + ci) * CHUNK
            pltpu.async_copy(src_hbm.at[pl.ds(base, CHUNK)], sidxs[b], swsem[b])
            pltpu.async_copy(w_hbm.at[pl.ds(base, CHUNK)], ws[b], swsem[b])

        def wait_sw(b):
            pltpu.make_async_copy(src_hbm.at[pl.ds(0, CHUNK)], sidxs[b], swsem[b]).wait()
            pltpu.make_async_copy(w_hbm.at[pl.ds(0, CHUNK)], ws[b], swsem[b]).wait()

        def start_d(ci, b):
            base = (cbase + ci) * CHUNK
            pltpu.async_copy(dst_hbm.at[pl.ds(base, CHUNK)], didxs[b], dsem[b])

        def wait_d(b):
            pltpu.make_async_copy(dst_hbm.at[pl.ds(0, CHUNK)], didxs[b], dsem[b]).wait()

        def start_gather(b):
            pltpu.async_copy(y_hbm.at[sidxs[b]], rows[b], gsem[b])

        def wait_gather(b):
            pltpu.make_async_copy(y_hbm.at[sidxs[b]], rows[b], gsem[b]).wait()

        def start_scatter(b):
            pltpu.async_copy(rows[b], acc_sh.at[didxs[b]], ssem[b], add=True)

        def wait_scatter(b):
            pltpu.make_async_copy(rows[b], acc_sh.at[didxs[b]], ssem[b]).wait()

        def scale(b):
            @pl.loop(0, CHUNK // 16)
            def _grp(k):
                wvec = ws[b][pl.ds(k * 16, 16)]
                for j in range(16):
                    wt = wvec[j]
                    r = k * 16 + j
                    for f in range(D // 16):
                        sl = pl.ds(f * 16, 16)
                        rows[b][r, sl] = rows[b][r, sl] * wt

        for b in range(NB):
            start_sw(b, b)
            start_d(b, b)
        for b in range(NB):
            wait_sw(b)
            start_gather(b)

        @pl.loop(0, CPW - NB, step=NB)
        def _super(ci0):
            for b in range(NB):
                wait_gather(b)
                scale(b)
                wait_d(b)
                start_scatter(b)
                start_sw(ci0 + NB + b, b)
            for b in range(NB):
                wait_scatter(b)
                start_d(ci0 + NB + b, b)
            for b in range(NB):
                wait_sw(b)
                start_gather(b)

        for b in range(NB):
            wait_gather(b)
            scale(b)
            wait_d(b)
            start_scatter(b)
        for b in range(NB):
            wait_scatter(b)

        plsc.subcore_barrier()

        for si, (rb, rs) in enumerate(ranges):
            if rs:
                @pl.when(s == si)
                def _(rb=rb, rs=rs):
                    pltpu.sync_copy(acc_sh.at[pl.ds(rb, rs)],
                                    z_hbm.at[c, pl.ds(rb, rs)])

    return run(y, ei[0], ei[1], w, zrows)


def _tc_matmul(x, W):
    def body(x_ref, w_ref, o_ref):
        o_ref[...] = jnp.dot(x_ref[...], w_ref[...],
                             preferred_element_type=jnp.float32)

    return pl.pallas_call(
        body,
        grid=(NBLK,),
        in_specs=[pl.BlockSpec((BR, D), lambda i: (i, 0)),
                  pl.BlockSpec((D, D), lambda i: (0, 0))],
        out_specs=pl.BlockSpec((BR, D), lambda i: (i, 0)),
        out_shape=jax.ShapeDtypeStruct((N, D), jnp.float32),
    )(x, W)


def _tc_layer(z0, z1, xin, b, Wroot, Wnext):
    def body(z0_ref, z1_ref, x_ref, b_ref, wr_ref, wn_ref, h_ref, y_ref):
        acc = (z0_ref[...] + z1_ref[...] + b_ref[...]
               + jnp.dot(x_ref[...], wr_ref[...],
                         preferred_element_type=jnp.float32))
        h = jnp.maximum(acc, 0.0)
        h_ref[...] = h
        y_ref[...] = jnp.dot(h, wn_ref[...], preferred_element_type=jnp.float32)

    return pl.pallas_call(
        body,
        grid=(NBLK,),
        in_specs=[pl.BlockSpec((BR, D), lambda i: (i, 0)),
                  pl.BlockSpec((BR, D), lambda i: (i, 0)),
                  pl.BlockSpec((BR, D), lambda i: (i, 0)),
                  pl.BlockSpec((1, D), lambda i: (0, 0)),
                  pl.BlockSpec((D, D), lambda i: (0, 0)),
                  pl.BlockSpec((D, D), lambda i: (0, 0))],
        out_specs=[pl.BlockSpec((BR, D), lambda i: (i, 0)),
                   pl.BlockSpec((BR, D), lambda i: (i, 0))],
        out_shape=[jax.ShapeDtypeStruct((N, D), jnp.float32),
                   jax.ShapeDtypeStruct((N, D), jnp.float32)],
    )(z0, z1, xin, b, Wroot, Wnext)


def _tc_layer3(z0, z1, h1, b2, W2root, W3rel, W3root, Wlin):
    def body(z0_ref, z1_ref, h1_ref, b_ref, w2r_ref, w3a_ref, w3b_ref,
             wl_ref, y3_ref, r_ref):
        acc = (z0_ref[...] + z1_ref[...] + b_ref[...]
               + jnp.dot(h1_ref[...], w2r_ref[...],
                         preferred_element_type=jnp.float32))
        h2 = jnp.maximum(acc, 0.0)
        y3_ref[...] = jnp.dot(h2, w3a_ref[...], preferred_element_type=jnp.float32)
        wr = jnp.dot(w3b_ref[...], wl_ref[...], preferred_element_type=jnp.float32)
        r_ref[...] = jnp.dot(h2, wr, preferred_element_type=jnp.float32)

    return pl.pallas_call(
        body,
        grid=(NBLK,),
        in_specs=[pl.BlockSpec((BR, D), lambda i: (i, 0)),
                  pl.BlockSpec((BR, D), lambda i: (i, 0)),
                  pl.BlockSpec((BR, D), lambda i: (i, 0)),
                  pl.BlockSpec((1, D), lambda i: (0, 0)),
                  pl.BlockSpec((D, D), lambda i: (0, 0)),
                  pl.BlockSpec((D, D), lambda i: (0, 0)),
                  pl.BlockSpec((D, D), lambda i: (0, 0)),
                  pl.BlockSpec((D, C), lambda i: (0, 0))],
        out_specs=[pl.BlockSpec((BR, D), lambda i: (i, 0)),
                   pl.BlockSpec((BR, C), lambda i: (i, 0))],
        out_shape=[jax.ShapeDtypeStruct((N, D), jnp.float32),
                   jax.ShapeDtypeStruct((N, C), jnp.float32)],
    )(z0, z1, h1, b2, W2root, W3rel, W3root, Wlin)


def _tc_final(z30, z31, batch2, r, b3, Wlin, blin):
    def body(z30_ref, z31_ref, b_ref, r_ref, b3_ref, wl_ref, bl_ref, o_ref,
             acc128_ref, acc_ref, cnt_ref):
        i = pl.program_id(0)

        @pl.when(i == 0)
        def _():
            acc128_ref[...] = jnp.zeros_like(acc128_ref)
            acc_ref[...] = jnp.zeros_like(acc_ref)
            cnt_ref[...] = jnp.zeros_like(cnt_ref)

        bv = b_ref[...]
        onehot = (bv == lax.broadcasted_iota(jnp.int32, (BR, G), 1)
                  ).astype(jnp.float32)
        acc128_ref[...] += lax.dot_general(
            onehot, z30_ref[...] + z31_ref[...], (((0,), (0,)), ((), ())),
            preferred_element_type=jnp.float32)
        acc_ref[...] += lax.dot_general(
            onehot, r_ref[...], (((0,), (0,)), ((), ())),
            preferred_element_type=jnp.float32)
        cnt_ref[...] += lax.dot_general(
            onehot, jnp.ones((BR, 1), jnp.float32), (((0,), (0,)), ((), ())),
            preferred_element_type=jnp.float32)

        @pl.when(i == NBLK - 1)
        def _():
            cnt = cnt_ref[...]
            b3w = jnp.dot(b3_ref[...], wl_ref[...],
                          preferred_element_type=jnp.float32)
            edge_term = jnp.dot(acc128_ref[...], wl_ref[...],
                                preferred_element_type=jnp.float32)
            tot = edge_term + acc_ref[...] + cnt * b3w
            logits = tot / jnp.maximum(cnt, 1.0) + bl_ref[...]
            m = jnp.max(logits, axis=-1, keepdims=True)
            ex = jnp.exp(logits - m)
            o_ref[...] = ex / jnp.sum(ex, axis=-1, keepdims=True)

    return pl.pallas_call(
        body,
        grid=(NBLK,),
        in_specs=[pl.BlockSpec((BR, D), lambda i: (i, 0)),
                  pl.BlockSpec((BR, D), lambda i: (i, 0)),
                  pl.BlockSpec((BR, 1), lambda i: (i, 0)),
                  pl.BlockSpec((BR, C), lambda i: (i, 0)),
                  pl.BlockSpec((1, D), lambda i: (0, 0)),
                  pl.BlockSpec((D, C), lambda i: (0, 0)),
                  pl.BlockSpec((1, C), lambda i: (0, 0))],
        out_specs=pl.BlockSpec((G, C), lambda i: (0, 0)),
        out_shape=jax.ShapeDtypeStruct((G, C), jnp.float32),
        scratch_shapes=[pltpu.VMEM((G, D), jnp.float32),
                        pltpu.VMEM((G, C), jnp.float32),
                        pltpu.VMEM((G, 1), jnp.float32)],
    )(z30, z31, batch2, r, b3, Wlin, blin)


def kernel(x, edge_attr, edge_index, batch,
           W1_rel, b1_rel, W1_root,
           W2_rel, b2_rel, W2_root,
           W3_rel, b3_rel, W3_root,
           W_lin, b_lin):
    pad = EP - E
    pad_idx = jnp.arange(pad, dtype=jnp.int32) % N
    src = jnp.concatenate([edge_index[0], pad_idx])
    dst = jnp.concatenate([edge_index[1], pad_idx])
    ei = jnp.stack([src, dst])
    w = jnp.concatenate([edge_attr, jnp.zeros((pad,), jnp.float32)])
    zrows = jnp.zeros((RPT_LAST, D), jnp.float32)
    batch2 = batch.reshape(N, 1)

    y1 = _tc_matmul(x, W1_rel)
    z1 = _sc_segsum(y1, ei, w, zrows, N)
    h1, y2 = _tc_layer(z1[0], z1[1], x, b1_rel.reshape(1, D), W1_root, W2_rel)
    z2 = _sc_segsum(y2, ei, w, zrows, N)
    y3, r = _tc_layer3(z2[0], z2[1], h1, b2_rel.reshape(1, D), W2_root,
                       W3_rel, W3_root, W_lin)
    z3 = _sc_segsum(y3, ei, w, zrows, N)
    return _tc_final(z3[0], z3[1], batch2, r, b3_rel.reshape(1, D), W_lin,
                     b_lin.reshape(1, C))


# speedup vs baseline: 1.0579x; 1.0335x over previous
import functools

import jax
import jax.numpy as jnp
from jax import lax
from jax.experimental import pallas as pl
from jax.experimental.pallas import tpu as pltpu
from jax.experimental.pallas import tpu_sc as plsc

N = 10000
D = 128
G = 64
C = 16
E = 320000

NC = 2
NS = 16
NW = NC * NS

CHUNK = 64
EP = 327680
CPW = EP // (NW * CHUNK)
RPT = 624
RPT_LAST = N - (NS - 1) * RPT

BR = 400
NBLK = N // BR


def _sc_mesh():
    return plsc.VectorSubcoreMesh(core_axis_name="c", subcore_axis_name="s")


def _sc_segsum(y, ei, w, zrows, nr):
    NB = 4
    if nr == N:
        ranges = [(si * RPT, RPT) for si in range(NS - 1)] + [((NS - 1) * RPT, RPT_LAST)]
    else:
        assert nr % 8 == 0 and nr // 8 <= NS
        ranges = [(si * 8, 8) if si < nr // 8 else (0, 0) for si in range(NS)]

    @functools.partial(
        pl.kernel,
        out_type=jax.ShapeDtypeStruct((NC, nr, D), jnp.float32),
        mesh=_sc_mesh(),
        scratch_types=(
            [pltpu.VMEM_SHARED((nr, D), jnp.float32)]
            + [pltpu.VMEM((CHUNK,), jnp.int32) for _ in range(NB)]
            + [pltpu.VMEM((CHUNK,), jnp.int32) for _ in range(NB)]
            + [pltpu.VMEM((CHUNK,), jnp.float32) for _ in range(NB)]
            + [pltpu.VMEM((CHUNK, D), jnp.float32) for _ in range(NB)]
            + [pltpu.SemaphoreType.DMA for _ in range(3 * NB)]
        ),
    )
    def run(y_hbm, src_hbm, dst_hbm, w_hbm, zr_hbm, z_hbm, acc_sh, *scr):
        sidxs = scr[0:NB]
        didxs = scr[NB:2 * NB]
        ws = scr[2 * NB:3 * NB]
        rows = scr[3 * NB:4 * NB]
        isem = scr[4 * NB:5 * NB]
        gsem = scr[5 * NB:6 * NB]
        ssem = scr[6 * NB:7 * NB]
        c = lax.axis_index("c")
        s = lax.axis_index("s")
        for si, (rb, rs) in enumerate(ranges):
            if rs:
                @pl.when(s == si)
                def _(rb=rb, rs=rs):
                    pltpu.sync_copy(zr_hbm.at[pl.ds(0, rs)],
                                    acc_sh.at[pl.ds(rb, rs)])

        plsc.subcore_barrier()
        wid = c * NS + s
        cbase = wid * CPW

        def start_idx(ci, b):
            base = (cbase + ci) * CHUNK
            pltpu.async_copy(src_hbm.at[pl.ds(base, CHUNK)], sidxs[b], isem[b])
            pltpu.async_copy(dst_hbm.at[pl.ds(base, CHUNK)], didxs[b], isem[b])
            pltpu.async_copy(w_hbm.at[pl.ds(base, CHUNK)], ws[b], isem[b])

        def wait_idx(b):
            pltpu.make_async_copy(src_hbm.at[pl.ds(0, CHUNK)], sidxs[b], isem[b]).wait()
            pltpu.make_async_copy(dst_hbm.at[pl.ds(0, CHUNK)], didxs[b], isem[b]).wait()
            pltpu.make_async_copy(w_hbm.at[pl.ds(0, CHUNK)], ws[b], isem[b]).wait()

        def start_gather(b):
            pltpu.async_copy(y_hbm.at[sidxs[b]], rows[b], gsem[b])

        def wait_gather(b):
            pltpu.make_async_copy(y_hbm.at[sidxs[b]], rows[b], gsem[b]).wait()

        def start_scatter(b):
            pltpu.async_copy(rows[b], acc_sh.at[didxs[b]], ssem[b], add=True)

        def wait_scatter(b):
            pltpu.make_async_copy(rows[b], acc_sh.at[didxs[b]], ssem[b]).wait()

        def scale(b):
            @pl.loop(0, CHUNK // 16)
            def _grp(k):
                wvec = ws[b][pl.ds(k * 16, 16)]
                for j in range(16):
                    wt = wvec[j]
                    r = k * 16 + j
                    for f in range(D // 16):
                        sl = pl.ds(f * 16, 16)
                        rows[b][r, sl] = rows[b][r, sl] * wt

        for b in range(NB):
            start_idx(b, b)
        for b in range(NB):
            wait_idx(b)
            start_gather(b)

        @pl.loop(0, CPW - NB, step=NB)
        def _super(ci0):
            for b in range(NB):
                wait_gather(b)
                scale(b)
                start_scatter(b)
            for b in range(NB):
                wait_scatter(b)
                start_idx(ci0 + NB + b, b)
            for b in range(NB):
                wait_idx(b)
                start_gather(b)

        for b in range(NB):
            wait_gather(b)
            scale(b)
            start_scatter(b)
        for b in range(NB):
            wait_scatter(b)

        plsc.subcore_barrier()

        for si, (rb, rs) in enumerate(ranges):
            if rs:
                @pl.when(s == si)
                def _(rb=rb, rs=rs):
                    pltpu.sync_copy(acc_sh.at[pl.ds(rb, rs)],
                                    z_hbm.at[c, pl.ds(rb, rs)])

    return run(y, ei[0], ei[1], w, zrows)


def _tc_matmul(x, W):
    def body(x_ref, w_ref, o_ref):
        o_ref[...] = jnp.dot(x_ref[...], w_ref[...],
                             preferred_element_type=jnp.float32)

    return pl.pallas_call(
        body,
        grid=(NBLK,),
        in_specs=[pl.BlockSpec((BR, D), lambda i: (i, 0)),
                  pl.BlockSpec((D, D), lambda i: (0, 0))],
        out_specs=pl.BlockSpec((BR, D), lambda i: (i, 0)),
        out_shape=jax.ShapeDtypeStruct((N, D), jnp.float32),
    )(x, W)


def _tc_layer(z0, z1, xin, b, Wroot, Wnext):
    def body(z0_ref, z1_ref, x_ref, b_ref, wr_ref, wn_ref, h_ref, y_ref):
        acc = (z0_ref[...] + z1_ref[...] + b_ref[...]
               + jnp.dot(x_ref[...], wr_ref[...],
                         preferred_element_type=jnp.float32))
        h = jnp.maximum(acc, 0.0)
        h_ref[...] = h
        y_ref[...] = jnp.dot(h, wn_ref[...], preferred_element_type=jnp.float32)

    return pl.pallas_call(
        body,
        grid=(NBLK,),
        in_specs=[pl.BlockSpec((BR, D), lambda i: (i, 0)),
                  pl.BlockSpec((BR, D), lambda i: (i, 0)),
                  pl.BlockSpec((BR, D), lambda i: (i, 0)),
                  pl.BlockSpec((1, D), lambda i: (0, 0)),
                  pl.BlockSpec((D, D), lambda i: (0, 0)),
                  pl.BlockSpec((D, D), lambda i: (0, 0))],
        out_specs=[pl.BlockSpec((BR, D), lambda i: (i, 0)),
                   pl.BlockSpec((BR, D), lambda i: (i, 0))],
        out_shape=[jax.ShapeDtypeStruct((N, D), jnp.float32),
                   jax.ShapeDtypeStruct((N, D), jnp.float32)],
    )(z0, z1, xin, b, Wroot, Wnext)


def _tc_layer3(z0, z1, h1, b2, W2root, W3rel, W3root, Wlin):
    def body(z0_ref, z1_ref, h1_ref, b_ref, w2r_ref, w3a_ref, w3b_ref,
             wl_ref, y3_ref, r_ref):
        acc = (z0_ref[...] + z1_ref[...] + b_ref[...]
               + jnp.dot(h1_ref[...], w2r_ref[...],
                         preferred_element_type=jnp.float32))
        h2 = jnp.maximum(acc, 0.0)
        y3_ref[...] = jnp.dot(h2, w3a_ref[...], preferred_element_type=jnp.float32)
        wr = jnp.dot(w3b_ref[...], wl_ref[...], preferred_element_type=jnp.float32)
        r_ref[...] = jnp.dot(h2, wr, preferred_element_type=jnp.float32)

    return pl.pallas_call(
        body,
        grid=(NBLK,),
        in_specs=[pl.BlockSpec((BR, D), lambda i: (i, 0)),
                  pl.BlockSpec((BR, D), lambda i: (i, 0)),
                  pl.BlockSpec((BR, D), lambda i: (i, 0)),
                  pl.BlockSpec((1, D), lambda i: (0, 0)),
                  pl.BlockSpec((D, D), lambda i: (0, 0)),
                  pl.BlockSpec((D, D), lambda i: (0, 0)),
                  pl.BlockSpec((D, D), lambda i: (0, 0)),
                  pl.BlockSpec((D, C), lambda i: (0, 0))],
        out_specs=[pl.BlockSpec((BR, D), lambda i: (i, 0)),
                   pl.BlockSpec((BR, C), lambda i: (i, 0))],
        out_shape=[jax.ShapeDtypeStruct((N, D), jnp.float32),
                   jax.ShapeDtypeStruct((N, C), jnp.float32)],
    )(z0, z1, h1, b2, W2root, W3rel, W3root, Wlin)


def _tc_final(z30, z31, batch2, r, b3, Wlin, blin):
    def body(z30_ref, z31_ref, b_ref, r_ref, b3_ref, wl_ref, bl_ref, o_ref,
             acc128_ref, acc_ref, cnt_ref):
        i = pl.program_id(0)

        @pl.when(i == 0)
        def _():
            acc128_ref[...] = jnp.zeros_like(acc128_ref)
            acc_ref[...] = jnp.zeros_like(acc_ref)
            cnt_ref[...] = jnp.zeros_like(cnt_ref)

        bv = b_ref[...]
        onehot = (bv == lax.broadcasted_iota(jnp.int32, (BR, G), 1)
                  ).astype(jnp.float32)
        acc128_ref[...] += lax.dot_general(
            onehot, z30_ref[...] + z31_ref[...], (((0,), (0,)), ((), ())),
            preferred_element_type=jnp.float32)
        acc_ref[...] += lax.dot_general(
            onehot, r_ref[...], (((0,), (0,)), ((), ())),
            preferred_element_type=jnp.float32)
        cnt_ref[...] += lax.dot_general(
            onehot, jnp.ones((BR, 1), jnp.float32), (((0,), (0,)), ((), ())),
            preferred_element_type=jnp.float32)

        @pl.when(i == NBLK - 1)
        def _():
            cnt = cnt_ref[...]
            b3w = jnp.dot(b3_ref[...], wl_ref[...],
                          preferred_element_type=jnp.float32)
            edge_term = jnp.dot(acc128_ref[...], wl_ref[...],
                                preferred_element_type=jnp.float32)
            tot = edge_term + acc_ref[...] + cnt * b3w
            logits = tot / jnp.maximum(cnt, 1.0) + bl_ref[...]
            m = jnp.max(logits, axis=-1, keepdims=True)
            ex = jnp.exp(logits - m)
            o_ref[...] = ex / jnp.sum(ex, axis=-1, keepdims=True)

    return pl.pallas_call(
        body,
        grid=(NBLK,),
        in_specs=[pl.BlockSpec((BR, D), lambda i: (i, 0)),
                  pl.BlockSpec((BR, D), lambda i: (i, 0)),
                  pl.BlockSpec((BR, 1), lambda i: (i, 0)),
                  pl.BlockSpec((BR, C), lambda i: (i, 0)),
                  pl.BlockSpec((1, D), lambda i: (0, 0)),
                  pl.BlockSpec((D, C), lambda i: (0, 0)),
                  pl.BlockSpec((1, C), lambda i: (0, 0))],
        out_specs=pl.BlockSpec((G, C), lambda i: (0, 0)),
        out_shape=jax.ShapeDtypeStruct((G, C), jnp.float32),
        scratch_shapes=[pltpu.VMEM((G, D), jnp.float32),
                        pltpu.VMEM((G, C), jnp.float32),
                        pltpu.VMEM((G, 1), jnp.float32)],
    )(z30, z31, batch2, r, b3, Wlin, blin)


def kernel(x, edge_attr, edge_index, batch,
           W1_rel, b1_rel, W1_root,
           W2_rel, b2_rel, W2_root,
           W3_rel, b3_rel, W3_root,
           W_lin, b_lin):
    pad = EP - E
    pad_idx = jnp.arange(pad, dtype=jnp.int32) % N
    src = jnp.concatenate([edge_index[0], pad_idx])
    dst = jnp.concatenate([edge_index[1], pad_idx])
    ei = jnp.stack([src, dst])
    w = jnp.concatenate([edge_attr, jnp.zeros((pad,), jnp.float32)])
    zrows = jnp.zeros((RPT_LAST, D), jnp.float32)
    batch2 = batch.reshape(N, 1)

    y1 = _tc_matmul(x, W1_rel)
    z1 = _sc_segsum(y1, ei, w, zrows, N)
    h1, y2 = _tc_layer(z1[0], z1[1], x, b1_rel.reshape(1, D), W1_root, W2_rel)
    z2 = _sc_segsum(y2, ei, w, zrows, N)
    y3, r = _tc_layer3(z2[0], z2[1], h1, b2_rel.reshape(1, D), W2_root,
                       W3_rel, W3_root, W_lin)
    z3 = _sc_segsum(y3, ei, w, zrows, N)
    return _tc_final(z3[0], z3[1], batch2, r, b3_rel.reshape(1, D), W_lin,
                     b_lin.reshape(1, C))


# speedup vs baseline: 1.0597x; 1.0017x over previous
import functools

import jax
import jax.numpy as jnp
from jax import lax
from jax.experimental import pallas as pl
from jax.experimental.pallas import tpu as pltpu
from jax.experimental.pallas import tpu_sc as plsc

N = 10000
D = 128
G = 64
C = 16
E = 320000

NC = 2
NS = 16
NW = NC * NS

CHUNK = 64
EP = 327680
CPW = EP // (NW * CHUNK)
RPT = 624
RPT_LAST = N - (NS - 1) * RPT

BR = 400
NBLK = N // BR


def _sc_mesh():
    return plsc.VectorSubcoreMesh(core_axis_name="c", subcore_axis_name="s")


def _sc_segsum(y, ei, w, zrows, nr):
    NB = 4
    if nr == N:
        ranges = [(si * RPT, RPT) for si in range(NS - 1)] + [((NS - 1) * RPT, RPT_LAST)]
    else:
        assert nr % 8 == 0 and nr // 8 <= NS
        ranges = [(si * 8, 8) if si < nr // 8 else (0, 0) for si in range(NS)]

    @functools.partial(
        pl.kernel,
        out_type=jax.ShapeDtypeStruct((NC, nr, D), jnp.float32),
        mesh=_sc_mesh(),
        scratch_types=(
            [pltpu.VMEM_SHARED((nr, D), jnp.float32)]
            + [pltpu.VMEM((CHUNK,), jnp.int32) for _ in range(NB)]
            + [pltpu.VMEM((CHUNK,), jnp.int32) for _ in range(NB)]
            + [pltpu.VMEM((CHUNK,), jnp.float32) for _ in range(NB)]
            + [pltpu.VMEM((CHUNK, D), jnp.float32) for _ in range(NB)]
            + [pltpu.SemaphoreType.DMA for _ in range(3 * NB)]
        ),
    )
    def run(y_hbm, src_hbm, dst_hbm, w_hbm, zr_hbm, z_hbm, acc_sh, *scr):
        sidxs = scr[0:NB]
        didxs = scr[NB:2 * NB]
        ws = scr[2 * NB:3 * NB]
        rows = scr[3 * NB:4 * NB]
        isem = scr[4 * NB:5 * NB]
        gsem = scr[5 * NB:6 * NB]
        ssem = scr[6 * NB:7 * NB]
        c = lax.axis_index("c")
        s = lax.axis_index("s")
        wid = c * NS + s
        cbase = wid * CPW
        def start_idx(ci, b):
            base = (cbase + ci) * CHUNK
            pltpu.async_copy(src_hbm.at[pl.ds(base, CHUNK)], sidxs[b], isem[b])
            pltpu.async_copy(dst_hbm.at[pl.ds(base, CHUNK)], didxs[b], isem[b])
            pltpu.async_copy(w_hbm.at[pl.ds(base, CHUNK)], ws[b], isem[b])

        def wait_idx(b):
            pltpu.make_async_copy(src_hbm.at[pl.ds(0, CHUNK)], sidxs[b], isem[b]).wait()
            pltpu.make_async_copy(dst_hbm.at[pl.ds(0, CHUNK)], didxs[b], isem[b]).wait()
            pltpu.make_async_copy(w_hbm.at[pl.ds(0, CHUNK)], ws[b], isem[b]).wait()

        def start_gather(b):
            pltpu.async_copy(y_hbm.at[sidxs[b]], rows[b], gsem[b])

        def wait_gather(b):
            pltpu.make_async_copy(y_hbm.at[sidxs[b]], rows[b], gsem[b]).wait()

        def start_scatter(b):
            pltpu.async_copy(rows[b], acc_sh.at[didxs[b]], ssem[b], add=True)

        def wait_scatter(b):
            pltpu.make_async_copy(rows[b], acc_sh.at[didxs[b]], ssem[b]).wait()

        def scale(b):
            @pl.loop(0, CHUNK // 16)
            def _grp(k):
                wvec = ws[b][pl.ds(k * 16, 16)]
                for j in range(16):
                    wt = wvec[j]
                    r = k * 16 + j
                    for f in range(D // 16):
                        sl = pl.ds(f * 16, 16)
                        rows[b][r, sl] = rows[b][r, sl] * wt

        for b in range(NB):
            start_idx(b, b)
        for b in range(NB):
            wait_idx(b)
            start_gather(b)

        for si, (rb, rs) in enumerate(ranges):
            if rs:
                @pl.when(s == si)
                def _(rb=rb, rs=rs):
                    pltpu.sync_copy(zr_hbm.at[pl.ds(0, rs)],
                                    acc_sh.at[pl.ds(rb, rs)])

        plsc.subcore_barrier()

        @pl.loop(0, CPW - NB, step=NB)
        def _super(ci0):
            for b in range(NB):
                wait_gather(b)
                scale(b)
                start_scatter(b)
            for b in range(NB):
                wait_scatter(b)
                start_idx(ci0 + NB + b, b)
            for b in range(NB):
                wait_idx(b)
                start_gather(b)

        for b in range(NB):
            wait_gather(b)
            scale(b)
            start_scatter(b)
        for b in range(NB):
            wait_scatter(b)

        plsc.subcore_barrier()

        for si, (rb, rs) in enumerate(ranges):
            if rs:
                @pl.when(s == si)
                def _(rb=rb, rs=rs):
                    pltpu.sync_copy(acc_sh.at[pl.ds(rb, rs)],
                                    z_hbm.at[c, pl.ds(rb, rs)])

    return run(y, ei[0], ei[1], w, zrows)


def _tc_matmul(x, W):
    def body(x_ref, w_ref, o_ref):
        o_ref[...] = jnp.dot(x_ref[...], w_ref[...],
                             preferred_element_type=jnp.float32)

    return pl.pallas_call(
        body,
        grid=(NBLK,),
        in_specs=[pl.BlockSpec((BR, D), lambda i: (i, 0)),
                  pl.BlockSpec((D, D), lambda i: (0, 0))],
        out_specs=pl.BlockSpec((BR, D), lambda i: (i, 0)),
        out_shape=jax.ShapeDtypeStruct((N, D), jnp.float32),
    )(x, W)


def _tc_layer(z0, z1, xin, b, Wroot, Wnext):
    def body(z0_ref, z1_ref, x_ref, b_ref, wr_ref, wn_ref, h_ref, y_ref):
        acc = (z0_ref[...] + z1_ref[...] + b_ref[...]
               + jnp.dot(x_ref[...], wr_ref[...],
                         preferred_element_type=jnp.float32))
        h = jnp.maximum(acc, 0.0)
        h_ref[...] = h
        y_ref[...] = jnp.dot(h, wn_ref[...], preferred_element_type=jnp.float32)

    return pl.pallas_call(
        body,
        grid=(NBLK,),
        in_specs=[pl.BlockSpec((BR, D), lambda i: (i, 0)),
                  pl.BlockSpec((BR, D), lambda i: (i, 0)),
                  pl.BlockSpec((BR, D), lambda i: (i, 0)),
                  pl.BlockSpec((1, D), lambda i: (0, 0)),
                  pl.BlockSpec((D, D), lambda i: (0, 0)),
                  pl.BlockSpec((D, D), lambda i: (0, 0))],
        out_specs=[pl.BlockSpec((BR, D), lambda i: (i, 0)),
                   pl.BlockSpec((BR, D), lambda i: (i, 0))],
        out_shape=[jax.ShapeDtypeStruct((N, D), jnp.float32),
                   jax.ShapeDtypeStruct((N, D), jnp.float32)],
    )(z0, z1, xin, b, Wroot, Wnext)


def _tc_layer3(z0, z1, h1, b2, W2root, W3rel, W3root, Wlin):
    def body(z0_ref, z1_ref, h1_ref, b_ref, w2r_ref, w3a_ref, w3b_ref,
             wl_ref, y3_ref, r_ref):
        acc = (z0_ref[...] + z1_ref[...] + b_ref[...]
               + jnp.dot(h1_ref[...], w2r_ref[...],
                         preferred_element_type=jnp.float32))
        h2 = jnp.maximum(acc, 0.0)
        y3_ref[...] = jnp.dot(h2, w3a_ref[...], preferred_element_type=jnp.float32)
        wr = jnp.dot(w3b_ref[...], wl_ref[...], preferred_element_type=jnp.float32)
        r_ref[...] = jnp.dot(h2, wr, preferred_element_type=jnp.float32)

    return pl.pallas_call(
        body,
        grid=(NBLK,),
        in_specs=[pl.BlockSpec((BR, D), lambda i: (i, 0)),
                  pl.BlockSpec((BR, D), lambda i: (i, 0)),
                  pl.BlockSpec((BR, D), lambda i: (i, 0)),
                  pl.BlockSpec((1, D), lambda i: (0, 0)),
                  pl.BlockSpec((D, D), lambda i: (0, 0)),
                  pl.BlockSpec((D, D), lambda i: (0, 0)),
                  pl.BlockSpec((D, D), lambda i: (0, 0)),
                  pl.BlockSpec((D, C), lambda i: (0, 0))],
        out_specs=[pl.BlockSpec((BR, D), lambda i: (i, 0)),
                   pl.BlockSpec((BR, C), lambda i: (i, 0))],
        out_shape=[jax.ShapeDtypeStruct((N, D), jnp.float32),
                   jax.ShapeDtypeStruct((N, C), jnp.float32)],
    )(z0, z1, h1, b2, W2root, W3rel, W3root, Wlin)


def _tc_final(z30, z31, batch2, r, b3, Wlin, blin):
    def body(z30_ref, z31_ref, b_ref, r_ref, b3_ref, wl_ref, bl_ref, o_ref,
             acc128_ref, acc_ref, cnt_ref):
        i = pl.program_id(0)

        @pl.when(i == 0)
        def _():
            acc128_ref[...] = jnp.zeros_like(acc128_ref)
            acc_ref[...] = jnp.zeros_like(acc_ref)
            cnt_ref[...] = jnp.zeros_like(cnt_ref)

        bv = b_ref[...]
        onehot = (bv == lax.broadcasted_iota(jnp.int32, (BR, G), 1)
                  ).astype(jnp.float32)
        acc128_ref[...] += lax.dot_general(
            onehot, z30_ref[...] + z31_ref[...], (((0,), (0,)), ((), ())),
            preferred_element_type=jnp.float32)
        acc_ref[...] += lax.dot_general(
            onehot, r_ref[...], (((0,), (0,)), ((), ())),
            preferred_element_type=jnp.float32)
        cnt_ref[...] += lax.dot_general(
            onehot, jnp.ones((BR, 1), jnp.float32), (((0,), (0,)), ((), ())),
            preferred_element_type=jnp.float32)

        @pl.when(i == NBLK - 1)
        def _():
            cnt = cnt_ref[...]
            b3w = jnp.dot(b3_ref[...], wl_ref[...],
                          preferred_element_type=jnp.float32)
            edge_term = jnp.dot(acc128_ref[...], wl_ref[...],
                                preferred_element_type=jnp.float32)
            tot = edge_term + acc_ref[...] + cnt * b3w
            logits = tot / jnp.maximum(cnt, 1.0) + bl_ref[...]
            m = jnp.max(logits, axis=-1, keepdims=True)
            ex = jnp.exp(logits - m)
            o_ref[...] = ex / jnp.sum(ex, axis=-1, keepdims=True)

    return pl.pallas_call(
        body,
        grid=(NBLK,),
        in_specs=[pl.BlockSpec((BR, D), lambda i: (i, 0)),
                  pl.BlockSpec((BR, D), lambda i: (i, 0)),
                  pl.BlockSpec((BR, 1), lambda i: (i, 0)),
                  pl.BlockSpec((BR, C), lambda i: (i, 0)),
                  pl.BlockSpec((1, D), lambda i: (0, 0)),
                  pl.BlockSpec((D, C), lambda i: (0, 0)),
                  pl.BlockSpec((1, C), lambda i: (0, 0))],
        out_specs=pl.BlockSpec((G, C), lambda i: (0, 0)),
        out_shape=jax.ShapeDtypeStruct((G, C), jnp.float32),
        scratch_shapes=[pltpu.VMEM((G, D), jnp.float32),
                        pltpu.VMEM((G, C), jnp.float32),
                        pltpu.VMEM((G, 1), jnp.float32)],
    )(z30, z31, batch2, r, b3, Wlin, blin)


def kernel(x, edge_attr, edge_index, batch,
           W1_rel, b1_rel, W1_root,
           W2_rel, b2_rel, W2_root,
           W3_rel, b3_rel, W3_root,
           W_lin, b_lin):
    pad = EP - E
    pad_idx = jnp.arange(pad, dtype=jnp.int32) % N
    src = jnp.concatenate([edge_index[0], pad_idx])
    dst = jnp.concatenate([edge_index[1], pad_idx])
    ei = jnp.stack([src, dst])
    w = jnp.concatenate([edge_attr, jnp.zeros((pad,), jnp.float32)])
    zrows = jnp.zeros((RPT_LAST, D), jnp.float32)
    batch2 = batch.reshape(N, 1)

    y1 = _tc_matmul(x, W1_rel)
    z1 = _sc_segsum(y1, ei, w, zrows, N)
    h1, y2 = _tc_layer(z1[0], z1[1], x, b1_rel.reshape(1, D), W1_root, W2_rel)
    z2 = _sc_segsum(y2, ei, w, zrows, N)
    y3, r = _tc_layer3(z2[0], z2[1], h1, b2_rel.reshape(1, D), W2_root,
                       W3_rel, W3_root, W_lin)
    z3 = _sc_segsum(y3, ei, w, zrows, N)
    return _tc_final(z3[0], z3[1], batch2, r, b3_rel.reshape(1, D), W_lin,
                     b_lin.reshape(1, C))
